# Initial kernel scaffold; baseline (speedup 1.0000x reference)
import jax
import jax.numpy as jnp
from jax.experimental import pallas as pl


def kernel(hidden_states, gating_output, topk, renormalize, out_gate_weight, stacked_in_gate_weights, stacked_in_up_weights):
    raise NotImplementedError("write your pallas kernel here")


# speedup vs baseline: 1.5060x; 1.5060x over previous
import functools

import jax
import jax.numpy as jnp
from jax.experimental import pallas as pl

_E = 8
_INNER = 8
_BSZ = 32
_OUT = _INNER * _BSZ
_HID = 1024
_TOK = 2048
_TILE_T = 256

_pallas_call = pl.pallas_call


def _router_body(x_ref, w_ref, w01_ref, e01_ref):
    x = x_ref[...]
    w = w_ref[...]
    logits = jax.lax.dot_general(
        x, w, (((1,), (1,)), ((), ())), preferred_element_type=jnp.float32)
    m = jnp.max(logits, axis=-1, keepdims=True)
    ex = jnp.exp(logits - m)
    sm = ex / jnp.sum(ex, axis=-1, keepdims=True)
    iot = jax.lax.broadcasted_iota(jnp.int32, sm.shape, 1)
    v0 = jnp.max(sm, axis=-1, keepdims=True)
    a0 = jnp.min(jnp.where(sm == v0, iot, _E), axis=-1, keepdims=True)
    sm1 = jnp.where(iot == a0, -1.0, sm)
    v1 = jnp.max(sm1, axis=-1, keepdims=True)
    a1 = jnp.min(jnp.where(sm1 == v1, iot, _E), axis=-1, keepdims=True)
    s = v0 + v1
    w01_ref[...] = jnp.where(iot == 0, v0 / s, jnp.where(iot == 1, v1 / s, 0.0))
    e01_ref[...] = jnp.where(iot == 0, a0, jnp.where(iot == 1, a1, 0))


def _scores_body(x_ref, wg_ref, wu_ref, avg_ref, s_ref):
    x = x_ref[...]
    wg = wg_ref[0]
    wu = wu_ref[0]
    g = jax.lax.dot_general(
        x, wg, (((1,), (1,)), ((), ())), preferred_element_type=jnp.float32)
    u = jax.lax.dot_general(
        x, wu, (((1,), (1,)), ((), ())), preferred_element_type=jnp.float32)
    p = jnp.abs(u * g * jax.nn.sigmoid(g))
    s_ref[0] = jax.lax.dot_general(
        p, avg_ref[...], (((1,), (0,)), ((), ())),
        preferred_element_type=jnp.float32)


def _select_body(s_ref, e01_ref, w01_ref, fw_ref, fid_ref):
    e01 = e01_ref[...]
    w01 = w01_ref[...]
    n = e01.shape[0]
    e0 = e01[:, 0:1]
    e1 = e01[:, 1:2]
    w0 = w01[:, 0:1]
    w1 = w01[:, 1:2]
    s0 = jnp.zeros((n, _INNER), jnp.float32)
    s1 = jnp.zeros((n, _INNER), jnp.float32)
    for e in range(_E):
        se = s_ref[e]
        s0 = jnp.where(e0 == e, se, s0)
        s1 = jnp.where(e1 == e, se, s1)
    jot = jax.lax.broadcasted_iota(jnp.int32, (n, _INNER), 1)

    def ranks(s):
        r = jnp.zeros((n, _INNER), jnp.int32)
        for jp in range(_INNER):
            c = s[:, jp:jp + 1]
            beat = (c > s) | ((c == s) & (jp < jot))
            r = r + beat.astype(jnp.int32)
        return r

    def desc_ids(sel, k):
        pos = jnp.zeros((n, _INNER), jnp.int32)
        for jp in range(_INNER):
            pos = pos + (sel[:, jp:jp + 1] & (jp > jot)).astype(jnp.int32)
        cols = []
        for m in range(k):
            hit = sel & (pos == m)
            cols.append(jnp.sum(jnp.where(hit, jot, 0), axis=1, keepdims=True))
        return jnp.concatenate(cols, axis=1)

    sel0 = ranks(s0) < 4
    sel1 = ranks(s1) < 2
    i0 = desc_ids(sel0, 4) + e0 * _INNER
    i1 = desc_ids(sel1, 2) + e1 * _INNER
    ids_a = jnp.concatenate([i0, i1], axis=1)
    ids_b = jnp.concatenate([i1, i0], axis=1)
    fid_ref[...] = jnp.where(e0 > e1, ids_a, ids_b)
    fw_ref[...] = jnp.concatenate(
        [jnp.broadcast_to(w0, (n, 4)), jnp.broadcast_to(w1, (n, 2))], axis=1)


def kernel(hidden_states, gating_output, topk, renormalize, out_gate_weight,
           stacked_in_gate_weights, stacked_in_up_weights):
    del gating_output, topk, renormalize
    x = hidden_states.astype(jnp.float32)
    n = x.shape[0]

    w01, e01 = _pallas_call(
        _router_body,
        out_shape=(
            jax.ShapeDtypeStruct((n, _E), jnp.float32),
            jax.ShapeDtypeStruct((n, _E), jnp.int32),
        ),
    )(x, out_gate_weight.astype(jnp.float32))

    avg = (jnp.equal(
        jnp.arange(_OUT)[:, None] // _BSZ,
        jnp.arange(_INNER)[None, :]).astype(jnp.float32) / _BSZ)

    n_t = n // _TILE_T
    scores = _pallas_call(
        _scores_body,
        grid=(_E, n_t),
        in_specs=[
            pl.BlockSpec((_TILE_T, _HID), lambda e, t: (t, 0)),
            pl.BlockSpec((1, _OUT, _HID), lambda e, t: (e, 0, 0)),
            pl.BlockSpec((1, _OUT, _HID), lambda e, t: (e, 0, 0)),
            pl.BlockSpec((_OUT, _INNER), lambda e, t: (0, 0)),
        ],
        out_specs=pl.BlockSpec((1, _TILE_T, _INNER), lambda e, t: (e, t, 0)),
        out_shape=jax.ShapeDtypeStruct((_E, n, _INNER), jnp.float32),
    )(x, stacked_in_gate_weights, stacked_in_up_weights, avg)

    fw, fid = _pallas_call(
        _select_body,
        out_shape=(
            jax.ShapeDtypeStruct((n, 6), jnp.float32),
            jax.ShapeDtypeStruct((n, 6), jnp.int32),
        ),
    )(scores, e01, w01)
    return fw, fid
